# Initial kernel scaffold; baseline (speedup 1.0000x reference)
import jax
import jax.numpy as jnp
from jax.experimental import pallas as pl


def kernel(x, edge_index, edge_attr, batch, pos, W_in, b_in, Wtp1, Wtp2, Wtp3, Wc1, bc1, Wc2, bc2, Wc3, bc3):
    raise NotImplementedError("write your pallas kernel here")


# speedup vs baseline: 10.6130x; 10.6130x over previous
import numpy as np
import jax
import jax.numpy as jnp
from jax import lax
from jax.experimental import pallas as pl
from jax.experimental.pallas import tpu as pltpu
from jax.experimental.pallas import tpu_sc as plsc

N = 10000
E = 320000
D = 128
H = 128
OUT = 10
G = 8

NC = 2
NS = 16
NW = NC * NS
EPW = E // NW
C = 80
CH = EPW // C
BI = 25
NB = 4
RCH = 400
NRCH = N // RCH
ZR = 40

K_SH = float(np.float32(0.28209479177387814) / np.float32(np.sqrt(np.float32(H))))

_f32 = jnp.float32


def _fill_vmem(ref, nrows, val16):
    def body(t, carry):
        ref[t // 8, pl.ds((t % 8) * 16, 16)] = val16
        return carry
    lax.fori_loop(0, nrows * 8, body, 0)


def _zero_chunks(zbuf, acc_sh, sid):
    def zero_chunk(base):
        for j in range(RCH // ZR):
            pltpu.sync_copy(zbuf, acc_sh.at[pl.ds(base + j * ZR, ZR)])

    zero_chunk(sid * RCH)

    @pl.when(sid < NRCH - NS)
    def _():
        zero_chunk((sid + NS) * RCH)


def _copy_chunks(acc_sh, out_hbm, cid, sid):
    def copy_chunk(base):
        sl = pl.ds(base, RCH)
        pltpu.sync_copy(acc_sh.at[sl], out_hbm.at[cid, sl])

    copy_chunk(sid * RCH)

    @pl.when(sid < NRCH - NS)
    def _():
        copy_chunk((sid + NS) * RCH)


def _make_sc_agg():
    mesh = plsc.VectorSubcoreMesh(core_axis_name="c", subcore_axis_name="s")

    NFULL = (BI // NB) * NB

    def body(h_hbm, row4_hbm, col4_hbm, part_out,
             ridx, cidx, rows_v, acc_sh, *gsems):
        cid = lax.axis_index("c")
        sid = lax.axis_index("s")
        wid = cid * NS + sid

        zslab = rows_v.at[0]
        _fill_vmem(zslab, C, jnp.zeros((16,), _f32))

        def zero_chunk(base):
            for j in range(RCH // C):
                pltpu.sync_copy(zslab, acc_sh.at[pl.ds(base + j * C, C)])

        zero_chunk(sid * RCH)

        @pl.when(sid < NRCH - NS)
        def _():
            zero_chunk((sid + NS) * RCH)
        plsc.subcore_barrier()

        def block(blk, carry):
            pltpu.sync_copy(row4_hbm.at[wid, blk], ridx)
            pltpu.sync_copy(col4_hbm.at[wid, blk], cidx)

            for b in range(NB):
                pltpu.async_copy(h_hbm.at[ridx.at[b]], rows_v.at[b], gsems[b])

            def outer(o, c2):
                for b in range(NB):
                    t = o * NB + b
                    pltpu.make_async_copy(
                        h_hbm.at[ridx.at[t]], rows_v.at[b], gsems[b]).wait()
                    pltpu.sync_copy(rows_v.at[b], acc_sh.at[cidx.at[t]],
                                    add=True)

                    @pl.when(t + NB < NFULL)
                    def _():
                        pltpu.async_copy(
                            h_hbm.at[ridx.at[t + NB]], rows_v.at[b], gsems[b])
                return c2

            lax.fori_loop(0, NFULL // NB, outer, 0)
            for t in range(NFULL, BI):
                pltpu.sync_copy(h_hbm.at[ridx.at[t]], rows_v.at[0])
                pltpu.sync_copy(rows_v.at[0], acc_sh.at[cidx.at[t]], add=True)
            return carry

        lax.fori_loop(0, CH // BI, block, 0)
        plsc.subcore_barrier()
        _copy_chunks(acc_sh, part_out, cid, sid)

    return pl.kernel(
        body,
        out_type=jax.ShapeDtypeStruct((NC, N, D), _f32),
        mesh=mesh,
        scratch_types=[
            pltpu.VMEM((BI, C), jnp.int32),
            pltpu.VMEM((BI, C), jnp.int32),
            pltpu.VMEM((NB, C, D), _f32),
            pltpu.VMEM_SHARED((N, D), _f32),
        ] + [pltpu.SemaphoreType.DMA] * NB,
    )


def _make_sc_cnt():
    mesh = plsc.VectorSubcoreMesh(core_axis_name="c", subcore_axis_name="s")

    def body(col3_hbm, cnt_out, cidx, ones_v, zbuf, acc_sh):
        cid = lax.axis_index("c")
        sid = lax.axis_index("s")
        wid = cid * NS + sid

        _fill_vmem(zbuf, ZR, jnp.zeros((16,), _f32))
        _fill_vmem(ones_v, C, jnp.ones((16,), _f32))
        _zero_chunks(zbuf, acc_sh, sid)
        pltpu.sync_copy(col3_hbm.at[wid], cidx)
        plsc.subcore_barrier()

        def step(t, carry):
            pltpu.sync_copy(ones_v, acc_sh.at[cidx.at[t]], add=True)
            return carry

        lax.fori_loop(0, CH, step, 0)
        plsc.subcore_barrier()
        _copy_chunks(acc_sh, cnt_out, cid, sid)

    return pl.kernel(
        body,
        out_type=jax.ShapeDtypeStruct((NC, N, D), _f32),
        mesh=mesh,
        scratch_types=[
            pltpu.VMEM((CH, C), jnp.int32),
            pltpu.VMEM((C, D), _f32),
            pltpu.VMEM((ZR, D), _f32),
            pltpu.VMEM_SHARED((N, D), _f32),
        ],
    )


def _mm_in(x, w, b):
    def body(x_ref, w_ref, b_ref, o_ref):
        acc = lax.dot_general(x_ref[...], w_ref[...], (((1,), (1,)), ((), ())),
                              preferred_element_type=_f32)
        o_ref[...] = jnp.maximum(acc + b_ref[...], 0.0)

    return pl.pallas_call(
        body, out_shape=jax.ShapeDtypeStruct((N, H), _f32),
    )(x, w, b.reshape(1, H))


def _mm_layer(part, cnt, w, do_relu):
    def body(p_ref, c_ref, w_ref, o_ref):
        p = p_ref[0] + p_ref[1]
        c = c_ref[0, :, 0:1] + c_ref[1, :, 0:1]
        inv = K_SH / jnp.maximum(c, 1.0)
        h = lax.dot_general(p * inv, w_ref[...], (((1,), (0,)), ((), ())),
                            preferred_element_type=_f32)
        o_ref[...] = jnp.maximum(h, 0.0) if do_relu else h

    return pl.pallas_call(
        body, out_shape=jax.ShapeDtypeStruct((N, H), _f32),
    )(part, cnt, w)


def _pool_mlp(h, batch2d, w1, b1, w2, b2, w3, b3):
    def body(h_ref, bt_ref, w1_ref, b1_ref, w2_ref, b2_ref, w3_ref, b3_ref,
             out_ref, z_ref):
        bt = bt_ref[...]
        groups = lax.broadcasted_iota(jnp.int32, (1, G), 1)
        oneh = (bt == groups).astype(_f32)
        zsum = lax.dot_general(oneh, h_ref[...], (((0,), (0,)), ((), ())),
                               preferred_element_type=_f32)
        ones = jnp.ones((N, 1), _f32)
        cnt = lax.dot_general(oneh, ones, (((0,), (0,)), ((), ())),
                              preferred_element_type=_f32)
        z = zsum / jnp.maximum(cnt, 1.0)
        h1 = jnp.maximum(
            lax.dot_general(z, w1_ref[...], (((1,), (1,)), ((), ())),
                            preferred_element_type=_f32) + b1_ref[...], 0.0)
        h2 = jnp.maximum(
            lax.dot_general(h1, w2_ref[...], (((1,), (1,)), ((), ())),
                            preferred_element_type=_f32) + b2_ref[...], 0.0)
        out = lax.dot_general(h2, w3_ref[...], (((1,), (1,)), ((), ())),
                              preferred_element_type=_f32) + b3_ref[...]
        out_ref[...] = out
        z_ref[...] = z

    return pl.pallas_call(
        body,
        out_shape=(jax.ShapeDtypeStruct((G, OUT), _f32),
                   jax.ShapeDtypeStruct((G, H), _f32)),
    )(h, batch2d, w1, b1.reshape(1, H), w2, b2.reshape(1, H), w3,
      b3.reshape(1, OUT))


def kernel(x, edge_index, edge_attr, batch, pos, W_in, b_in,
           Wtp1, Wtp2, Wtp3, Wc1, bc1, Wc2, bc2, Wc3, bc3):
    row4 = edge_index[0].astype(jnp.int32).reshape(NW, CH // BI, BI, C)
    col4 = edge_index[1].astype(jnp.int32).reshape(NW, CH // BI, BI, C)
    col3 = edge_index[1].astype(jnp.int32).reshape(NW, CH, C)

    h0 = _mm_in(x, W_in, b_in)

    agg = _make_sc_agg()
    cnt = _make_sc_cnt()(col3)

    part1 = agg(h0, row4, col4)
    h1 = _mm_layer(part1, cnt, Wtp1, True)
    part2 = agg(h1, row4, col4)
    h2 = _mm_layer(part2, cnt, Wtp2, True)
    part3 = agg(h2, row4, col4)
    h3 = _mm_layer(part3, cnt, Wtp3, False)

    out, z = _pool_mlp(h3, batch.astype(jnp.int32).reshape(N, 1),
                       Wc1, bc1, Wc2, bc2, Wc3, bc3)
    return (out, z)


# speedup vs baseline: 10.6663x; 1.0050x over previous
import numpy as np
import jax
import jax.numpy as jnp
from jax import lax
from jax.experimental import pallas as pl
from jax.experimental.pallas import tpu as pltpu
from jax.experimental.pallas import tpu_sc as plsc

N = 10000
E = 320000
D = 128
H = 128
OUT = 10
G = 8

NC = 2
NS = 16
NW = NC * NS
EPW = E // NW
C = 80
CH = EPW // C
BI = 25
NB = 4
RCH = 400
NRCH = N // RCH
ZR = 40

K_SH = float(np.float32(0.28209479177387814) / np.float32(np.sqrt(np.float32(H))))

_f32 = jnp.float32


def _fill_vmem(ref, nrows, val16):
    def body(t, carry):
        ref[t // 8, pl.ds((t % 8) * 16, 16)] = val16
        return carry
    lax.fori_loop(0, nrows * 8, body, 0)


def _zero_chunks(zbuf, acc_sh, sid):
    def zero_chunk(base):
        for j in range(RCH // ZR):
            pltpu.sync_copy(zbuf, acc_sh.at[pl.ds(base + j * ZR, ZR)])

    zero_chunk(sid * RCH)

    @pl.when(sid < NRCH - NS)
    def _():
        zero_chunk((sid + NS) * RCH)


def _copy_chunks(acc_sh, out_hbm, cid, sid):
    def copy_chunk(base):
        sl = pl.ds(base, RCH)
        pltpu.sync_copy(acc_sh.at[sl], out_hbm.at[cid, sl])

    copy_chunk(sid * RCH)

    @pl.when(sid < NRCH - NS)
    def _():
        copy_chunk((sid + NS) * RCH)


def _make_sc_agg():
    mesh = plsc.VectorSubcoreMesh(core_axis_name="c", subcore_axis_name="s")

    NFULL = (BI // NB) * NB

    def body(h_hbm, row4_hbm, col4_hbm, part_out,
             ridx, cidx, rows_v, acc_sh, *gsems):
        cid = lax.axis_index("c")
        sid = lax.axis_index("s")
        wid = cid * NS + sid

        zslab = rows_v.at[0]
        _fill_vmem(zslab, C, jnp.zeros((16,), _f32))

        def zero_chunk(base):
            for j in range(RCH // C):
                pltpu.sync_copy(zslab, acc_sh.at[pl.ds(base + j * C, C)])

        zero_chunk(sid * RCH)

        @pl.when(sid < NRCH - NS)
        def _():
            zero_chunk((sid + NS) * RCH)
        plsc.subcore_barrier()

        def block(blk, carry):
            pltpu.sync_copy(row4_hbm.at[wid, blk], ridx)
            pltpu.sync_copy(col4_hbm.at[wid, blk], cidx)

            for b in range(NB):
                pltpu.async_copy(h_hbm.at[ridx.at[b]], rows_v.at[b], gsems[b])

            def outer(o, c2):
                for b in range(NB):
                    t = o * NB + b
                    pltpu.make_async_copy(
                        h_hbm.at[ridx.at[t]], rows_v.at[b], gsems[b]).wait()
                    pltpu.sync_copy(rows_v.at[b], acc_sh.at[cidx.at[t]],
                                    add=True)

                    @pl.when(t + NB < NFULL)
                    def _():
                        pltpu.async_copy(
                            h_hbm.at[ridx.at[t + NB]], rows_v.at[b], gsems[b])
                return c2

            lax.fori_loop(0, NFULL // NB, outer, 0)
            for t in range(NFULL, BI):
                pltpu.sync_copy(h_hbm.at[ridx.at[t]], rows_v.at[0])
                pltpu.sync_copy(rows_v.at[0], acc_sh.at[cidx.at[t]], add=True)
            return carry

        lax.fori_loop(0, CH // BI, block, 0)
        plsc.subcore_barrier()
        _copy_chunks(acc_sh, part_out, cid, sid)

    return pl.kernel(
        body,
        out_type=jax.ShapeDtypeStruct((NC, N, D), _f32),
        mesh=mesh,
        scratch_types=[
            pltpu.VMEM((BI, C), jnp.int32),
            pltpu.VMEM((BI, C), jnp.int32),
            pltpu.VMEM((NB, C, D), _f32),
            pltpu.VMEM_SHARED((N, D), _f32),
        ] + [pltpu.SemaphoreType.DMA] * NB,
    )


def _make_sc_cnt():
    mesh = plsc.VectorSubcoreMesh(core_axis_name="c", subcore_axis_name="s")

    def body(col3_hbm, cnt_out, cidx, ones_v, zbuf, acc_sh):
        cid = lax.axis_index("c")
        sid = lax.axis_index("s")
        wid = cid * NS + sid

        _fill_vmem(zbuf, ZR, jnp.zeros((16,), _f32))
        _fill_vmem(ones_v, C, jnp.ones((16,), _f32))
        _zero_chunks(zbuf, acc_sh, sid)
        pltpu.sync_copy(col3_hbm.at[wid], cidx)
        plsc.subcore_barrier()

        def step(t, carry):
            pltpu.sync_copy(ones_v, acc_sh.at[cidx.at[t]], add=True)
            return carry

        lax.fori_loop(0, CH, step, 0)
        plsc.subcore_barrier()
        _copy_chunks(acc_sh, cnt_out, cid, sid)

    return pl.kernel(
        body,
        out_type=jax.ShapeDtypeStruct((NC, N, D), _f32),
        mesh=mesh,
        scratch_types=[
            pltpu.VMEM((CH, C), jnp.int32),
            pltpu.VMEM((C, D), _f32),
            pltpu.VMEM((ZR, D), _f32),
            pltpu.VMEM_SHARED((N, D), _f32),
        ],
    )


def _inv_kernel(cnt):
    def body(c_ref, o_ref):
        c = c_ref[0, :, 0:1] + c_ref[1, :, 0:1]
        o_ref[...] = K_SH / jnp.maximum(c, 1.0)

    return pl.pallas_call(
        body, out_shape=jax.ShapeDtypeStruct((N, 1), _f32),
    )(cnt)


def _mm_in(x, w, b, dep):
    def body(x_ref, w_ref, b_ref, d_ref, o_ref):
        del d_ref
        acc = lax.dot_general(x_ref[...], w_ref[...], (((1,), (1,)), ((), ())),
                              preferred_element_type=_f32)
        o_ref[...] = jnp.maximum(acc + b_ref[...], 0.0)

    return pl.pallas_call(
        body, out_shape=jax.ShapeDtypeStruct((N, H), _f32),
    )(x, w, b.reshape(1, H), dep)


def _mm_layer(part, inv, w):
    def body(p_ref, i_ref, w_ref, o_ref):
        p = p_ref[0] + p_ref[1]
        h = lax.dot_general(p * i_ref[...], w_ref[...],
                            (((1,), (0,)), ((), ())),
                            preferred_element_type=_f32)
        o_ref[...] = jnp.maximum(h, 0.0)

    return pl.pallas_call(
        body, out_shape=jax.ShapeDtypeStruct((N, H), _f32),
    )(part, inv, w)


def _pool_mlp(part3, inv, wtp3, batch2d, w1, b1, w2, b2, w3, b3):
    def body(p_ref, i_ref, wtp_ref, bt_ref, w1_ref, b1_ref, w2_ref, b2_ref,
             w3_ref, b3_ref, out_ref, z_ref):
        p = p_ref[0] + p_ref[1]
        h3 = lax.dot_general(p * i_ref[...], wtp_ref[...],
                             (((1,), (0,)), ((), ())),
                             preferred_element_type=_f32)
        bt = bt_ref[...]
        groups = lax.broadcasted_iota(jnp.int32, (1, G), 1)
        oneh = (bt == groups).astype(_f32)
        zsum = lax.dot_general(oneh, h3, (((0,), (0,)), ((), ())),
                               preferred_element_type=_f32)
        ones = jnp.ones((N, 1), _f32)
        cnt = lax.dot_general(oneh, ones, (((0,), (0,)), ((), ())),
                              preferred_element_type=_f32)
        z = zsum / jnp.maximum(cnt, 1.0)
        h1 = jnp.maximum(
            lax.dot_general(z, w1_ref[...], (((1,), (1,)), ((), ())),
                            preferred_element_type=_f32) + b1_ref[...], 0.0)
        h2 = jnp.maximum(
            lax.dot_general(h1, w2_ref[...], (((1,), (1,)), ((), ())),
                            preferred_element_type=_f32) + b2_ref[...], 0.0)
        out = lax.dot_general(h2, w3_ref[...], (((1,), (1,)), ((), ())),
                              preferred_element_type=_f32) + b3_ref[...]
        out_ref[...] = out
        z_ref[...] = z

    return pl.pallas_call(
        body,
        out_shape=(jax.ShapeDtypeStruct((G, OUT), _f32),
                   jax.ShapeDtypeStruct((G, H), _f32)),
    )(part3, inv, wtp3, batch2d, w1, b1.reshape(1, H), w2, b2.reshape(1, H),
      w3, b3.reshape(1, OUT))


def kernel(x, edge_index, edge_attr, batch, pos, W_in, b_in,
           Wtp1, Wtp2, Wtp3, Wc1, bc1, Wc2, bc2, Wc3, bc3):
    row4 = edge_index[0].astype(jnp.int32).reshape(NW, CH // BI, BI, C)
    col4 = edge_index[1].astype(jnp.int32).reshape(NW, CH // BI, BI, C)
    col3 = edge_index[1].astype(jnp.int32).reshape(NW, CH, C)

    agg = _make_sc_agg()
    inv = _inv_kernel(_make_sc_cnt()(col3))
    h0 = _mm_in(x, W_in, b_in, inv)

    part1 = agg(h0, row4, col4)
    h1 = _mm_layer(part1, inv, Wtp1)
    part2 = agg(h1, row4, col4)
    h2 = _mm_layer(part2, inv, Wtp2)
    part3 = agg(h2, row4, col4)

    out, z = _pool_mlp(part3, inv, Wtp3,
                       batch.astype(jnp.int32).reshape(N, 1),
                       Wc1, bc1, Wc2, bc2, Wc3, bc3)
    return (out, z)


# speedup vs baseline: 10.7081x; 1.0039x over previous
import numpy as np
import jax
import jax.numpy as jnp
from jax import lax
from jax.experimental import pallas as pl
from jax.experimental.pallas import tpu as pltpu
from jax.experimental.pallas import tpu_sc as plsc

N = 10000
E = 320000
D = 128
H = 128
OUT = 10
G = 8

NC = 2
NS = 16
NW = NC * NS
EPW = E // NW
C = 80
CH = EPW // C
BI = 25
NB = 4
RCH = 400
NRCH = N // RCH
ZR = 40

K_SH = float(np.float32(0.28209479177387814) / np.float32(np.sqrt(np.float32(H))))

_f32 = jnp.float32


def _fill_vmem(ref, nrows, val16):
    def body(t, carry):
        ref[t // 8, pl.ds((t % 8) * 16, 16)] = val16
        return carry
    lax.fori_loop(0, nrows * 8, body, 0)


def _zero_chunks(zbuf, acc_sh, sid):
    def zero_chunk(base):
        for j in range(RCH // ZR):
            pltpu.sync_copy(zbuf, acc_sh.at[pl.ds(base + j * ZR, ZR)])

    zero_chunk(sid * RCH)

    @pl.when(sid < NRCH - NS)
    def _():
        zero_chunk((sid + NS) * RCH)


def _copy_chunks(acc_sh, out_hbm, cid, sid):
    def copy_chunk(base):
        sl = pl.ds(base, RCH)
        pltpu.sync_copy(acc_sh.at[sl], out_hbm.at[cid, sl])

    copy_chunk(sid * RCH)

    @pl.when(sid < NRCH - NS)
    def _():
        copy_chunk((sid + NS) * RCH)


def _make_sc_agg():
    mesh = plsc.VectorSubcoreMesh(core_axis_name="c", subcore_axis_name="s")

    NFULL = (BI // NB) * NB

    def body(h_hbm, row4_hbm, col4_hbm, part_out,
             ridx, cidx, rows_v, acc_sh, *gsems):
        cid = lax.axis_index("c")
        sid = lax.axis_index("s")
        wid = cid * NS + sid

        zslab = rows_v.at[0]
        _fill_vmem(zslab, C, jnp.zeros((16,), _f32))

        def zero_chunk(base):
            for j in range(RCH // C):
                pltpu.sync_copy(zslab, acc_sh.at[pl.ds(base + j * C, C)])

        zero_chunk(sid * RCH)

        @pl.when(sid < NRCH - NS)
        def _():
            zero_chunk((sid + NS) * RCH)
        plsc.subcore_barrier()

        def block(blk, carry):
            pltpu.sync_copy(row4_hbm.at[wid, blk], ridx)
            pltpu.sync_copy(col4_hbm.at[wid, blk], cidx)

            for b in range(NB):
                pltpu.async_copy(h_hbm.at[ridx.at[b]], rows_v.at[b], gsems[b])

            def outer(o, c2):
                for b in range(NB):
                    t = o * NB + b
                    pltpu.make_async_copy(
                        h_hbm.at[ridx.at[t]], rows_v.at[b], gsems[b]).wait()
                    pltpu.sync_copy(rows_v.at[b], acc_sh.at[cidx.at[t]],
                                    add=True)

                    @pl.when(t + NB < NFULL)
                    def _():
                        pltpu.async_copy(
                            h_hbm.at[ridx.at[t + NB]], rows_v.at[b], gsems[b])
                return c2

            lax.fori_loop(0, NFULL // NB, outer, 0)
            for t in range(NFULL, BI):
                pltpu.sync_copy(h_hbm.at[ridx.at[t]], rows_v.at[0])
                pltpu.sync_copy(rows_v.at[0], acc_sh.at[cidx.at[t]], add=True)
            return carry

        lax.fori_loop(0, CH // BI, block, 0)
        plsc.subcore_barrier()
        _copy_chunks(acc_sh, part_out, cid, sid)

    return pl.kernel(
        body,
        out_type=jax.ShapeDtypeStruct((NC, N, D), _f32),
        mesh=mesh,
        scratch_types=[
            pltpu.VMEM((BI, C), jnp.int32),
            pltpu.VMEM((BI, C), jnp.int32),
            pltpu.VMEM((NB, C, D), _f32),
            pltpu.VMEM_SHARED((N, D), _f32),
        ] + [pltpu.SemaphoreType.DMA] * NB,
    )


def _make_sc_cnt():
    mesh = plsc.VectorSubcoreMesh(core_axis_name="c", subcore_axis_name="s")

    NSS = 4
    NFULL = (CH // NSS) * NSS

    def body(col3_hbm, cnt_out, cidx, ones_v, zbuf, acc_sh, *ssems):
        cid = lax.axis_index("c")
        sid = lax.axis_index("s")
        wid = cid * NS + sid

        _fill_vmem(zbuf, ZR, jnp.zeros((16,), _f32))
        _fill_vmem(ones_v, C, jnp.ones((16,), _f32))
        _zero_chunks(zbuf, acc_sh, sid)
        pltpu.sync_copy(col3_hbm.at[wid], cidx)
        plsc.subcore_barrier()

        for b in range(NSS):
            pltpu.async_copy(ones_v, acc_sh.at[cidx.at[b]], ssems[b], add=True)

        def outer(o, carry):
            for b in range(NSS):
                t = o * NSS + b
                pltpu.make_async_copy(
                    ones_v, acc_sh.at[cidx.at[t]], ssems[b]).wait()

                @pl.when(t + NSS < NFULL)
                def _():
                    pltpu.async_copy(ones_v, acc_sh.at[cidx.at[t + NSS]],
                                     ssems[b], add=True)
            return carry

        lax.fori_loop(0, NFULL // NSS, outer, 0)
        for t in range(NFULL, CH):
            pltpu.sync_copy(ones_v, acc_sh.at[cidx.at[t]], add=True)
        plsc.subcore_barrier()
        _copy_chunks(acc_sh, cnt_out, cid, sid)

    return pl.kernel(
        body,
        out_type=jax.ShapeDtypeStruct((NC, N, D), _f32),
        mesh=mesh,
        scratch_types=[
            pltpu.VMEM((CH, C), jnp.int32),
            pltpu.VMEM((C, D), _f32),
            pltpu.VMEM((ZR, D), _f32),
            pltpu.VMEM_SHARED((N, D), _f32),
        ] + [pltpu.SemaphoreType.DMA] * NSS,
    )


def _inv_kernel(cnt):
    def body(c_ref, o_ref):
        c = c_ref[0, :, 0:1] + c_ref[1, :, 0:1]
        o_ref[...] = K_SH / jnp.maximum(c, 1.0)

    return pl.pallas_call(
        body, out_shape=jax.ShapeDtypeStruct((N, 1), _f32),
    )(cnt)


def _mm_in(x, w, b, dep):
    def body(x_ref, w_ref, b_ref, d_ref, o_ref):
        del d_ref
        acc = lax.dot_general(x_ref[...], w_ref[...], (((1,), (1,)), ((), ())),
                              preferred_element_type=_f32)
        o_ref[...] = jnp.maximum(acc + b_ref[...], 0.0)

    return pl.pallas_call(
        body, out_shape=jax.ShapeDtypeStruct((N, H), _f32),
    )(x, w, b.reshape(1, H), dep)


def _mm_layer(part, inv, w):
    def body(p_ref, i_ref, w_ref, o_ref):
        p = p_ref[0] + p_ref[1]
        h = lax.dot_general(p * i_ref[...], w_ref[...],
                            (((1,), (0,)), ((), ())),
                            preferred_element_type=_f32)
        o_ref[...] = jnp.maximum(h, 0.0)

    return pl.pallas_call(
        body, out_shape=jax.ShapeDtypeStruct((N, H), _f32),
    )(part, inv, w)


def _pool_mlp(part3, inv, wtp3, batch2d, w1, b1, w2, b2, w3, b3):
    def body(p_ref, i_ref, wtp_ref, bt_ref, w1_ref, b1_ref, w2_ref, b2_ref,
             w3_ref, b3_ref, out_ref, z_ref):
        p = p_ref[0] + p_ref[1]
        h3 = lax.dot_general(p * i_ref[...], wtp_ref[...],
                             (((1,), (0,)), ((), ())),
                             preferred_element_type=_f32)
        bt = bt_ref[...]
        groups = lax.broadcasted_iota(jnp.int32, (1, G), 1)
        oneh = (bt == groups).astype(_f32)
        zsum = lax.dot_general(oneh, h3, (((0,), (0,)), ((), ())),
                               preferred_element_type=_f32)
        ones = jnp.ones((N, 1), _f32)
        cnt = lax.dot_general(oneh, ones, (((0,), (0,)), ((), ())),
                              preferred_element_type=_f32)
        z = zsum / jnp.maximum(cnt, 1.0)
        h1 = jnp.maximum(
            lax.dot_general(z, w1_ref[...], (((1,), (1,)), ((), ())),
                            preferred_element_type=_f32) + b1_ref[...], 0.0)
        h2 = jnp.maximum(
            lax.dot_general(h1, w2_ref[...], (((1,), (1,)), ((), ())),
                            preferred_element_type=_f32) + b2_ref[...], 0.0)
        out = lax.dot_general(h2, w3_ref[...], (((1,), (1,)), ((), ())),
                              preferred_element_type=_f32) + b3_ref[...]
        out_ref[...] = out
        z_ref[...] = z

    return pl.pallas_call(
        body,
        out_shape=(jax.ShapeDtypeStruct((G, OUT), _f32),
                   jax.ShapeDtypeStruct((G, H), _f32)),
    )(part3, inv, wtp3, batch2d, w1, b1.reshape(1, H), w2, b2.reshape(1, H),
      w3, b3.reshape(1, OUT))


def kernel(x, edge_index, edge_attr, batch, pos, W_in, b_in,
           Wtp1, Wtp2, Wtp3, Wc1, bc1, Wc2, bc2, Wc3, bc3):
    row4 = edge_index[0].astype(jnp.int32).reshape(NW, CH // BI, BI, C)
    col4 = edge_index[1].astype(jnp.int32).reshape(NW, CH // BI, BI, C)
    col3 = edge_index[1].astype(jnp.int32).reshape(NW, CH, C)

    agg = _make_sc_agg()
    inv = _inv_kernel(_make_sc_cnt()(col3))
    h0 = _mm_in(x, W_in, b_in, inv)

    part1 = agg(h0, row4, col4)
    h1 = _mm_layer(part1, inv, Wtp1)
    part2 = agg(h1, row4, col4)
    h2 = _mm_layer(part2, inv, Wtp2)
    part3 = agg(h2, row4, col4)

    out, z = _pool_mlp(part3, inv, Wtp3,
                       batch.astype(jnp.int32).reshape(N, 1),
                       Wc1, bc1, Wc2, bc2, Wc3, bc3)
    return (out, z)


# speedup vs baseline: 10.9083x; 1.0187x over previous
import numpy as np
import jax
import jax.numpy as jnp
from jax import lax
from jax.experimental import pallas as pl
from jax.experimental.pallas import tpu as pltpu
from jax.experimental.pallas import tpu_sc as plsc

N = 10000
E = 320000
D = 128
H = 128
OUT = 10
G = 8

NC = 2
NS = 16
NW = NC * NS
EPW = E // NW
C = 80
CH = EPW // C
BI = 25
NB = 4
RCH = 400
NRCH = N // RCH
ZR = 40

K_SH = float(np.float32(0.28209479177387814) / np.float32(np.sqrt(np.float32(H))))

_f32 = jnp.float32


def _fill_vmem(ref, nrows, val16):
    def body(t, carry):
        ref[t // 8, pl.ds((t % 8) * 16, 16)] = val16
        return carry
    lax.fori_loop(0, nrows * 8, body, 0)


def _zero_chunks(zbuf, acc_sh, sid):
    def zero_chunk(base):
        for j in range(RCH // ZR):
            pltpu.sync_copy(zbuf, acc_sh.at[pl.ds(base + j * ZR, ZR)])

    zero_chunk(sid * RCH)

    @pl.when(sid < NRCH - NS)
    def _():
        zero_chunk((sid + NS) * RCH)


def _copy_chunks(acc_sh, out_hbm, cid, sid):
    def copy_chunk(base):
        sl = pl.ds(base, RCH)
        pltpu.sync_copy(acc_sh.at[sl], out_hbm.at[cid, sl])

    copy_chunk(sid * RCH)

    @pl.when(sid < NRCH - NS)
    def _():
        copy_chunk((sid + NS) * RCH)


def _make_sc_agg():
    mesh = plsc.VectorSubcoreMesh(core_axis_name="c", subcore_axis_name="s")

    NFULL = (BI // NB) * NB

    def body(h_hbm, row4_hbm, col4_hbm, dep_hbm, part_out,
             ridx, cidx, rows_v, acc_sh, *gsems):
        del dep_hbm
        cid = lax.axis_index("c")
        sid = lax.axis_index("s")
        wid = cid * NS + sid

        zslab = rows_v.at[0]
        _fill_vmem(zslab, C, jnp.zeros((16,), _f32))

        def zero_chunk(base):
            for j in range(RCH // C):
                pltpu.sync_copy(zslab, acc_sh.at[pl.ds(base + j * C, C)])

        zero_chunk(sid * RCH)

        @pl.when(sid < NRCH - NS)
        def _():
            zero_chunk((sid + NS) * RCH)
        plsc.subcore_barrier()

        def block(blk, carry):
            pltpu.sync_copy(row4_hbm.at[wid, blk], ridx)
            pltpu.sync_copy(col4_hbm.at[wid, blk], cidx)

            for b in range(NB):
                pltpu.async_copy(h_hbm.at[ridx.at[b]], rows_v.at[b], gsems[b])

            def outer(o, c2):
                for b in range(NB):
                    t = o * NB + b
                    pltpu.make_async_copy(
                        h_hbm.at[ridx.at[t]], rows_v.at[b], gsems[b]).wait()
                    pltpu.sync_copy(rows_v.at[b], acc_sh.at[cidx.at[t]],
                                    add=True)

                    @pl.when(t + NB < NFULL)
                    def _():
                        pltpu.async_copy(
                            h_hbm.at[ridx.at[t + NB]], rows_v.at[b], gsems[b])
                return c2

            lax.fori_loop(0, NFULL // NB, outer, 0)
            for t in range(NFULL, BI):
                pltpu.sync_copy(h_hbm.at[ridx.at[t]], rows_v.at[0])
                pltpu.sync_copy(rows_v.at[0], acc_sh.at[cidx.at[t]], add=True)
            return carry

        lax.fori_loop(0, CH // BI, block, 0)
        plsc.subcore_barrier()
        _copy_chunks(acc_sh, part_out, cid, sid)

    return pl.kernel(
        body,
        out_type=jax.ShapeDtypeStruct((NC, N, D), _f32),
        mesh=mesh,
        scratch_types=[
            pltpu.VMEM((BI, C), jnp.int32),
            pltpu.VMEM((BI, C), jnp.int32),
            pltpu.VMEM((NB, C, D), _f32),
            pltpu.VMEM_SHARED((N, D), _f32),
        ] + [pltpu.SemaphoreType.DMA] * NB,
    )


def _make_sc_cnt():
    mesh = plsc.VectorSubcoreMesh(core_axis_name="c", subcore_axis_name="s")

    NSS = 4
    NFULL = (CH // NSS) * NSS

    def body(col3_hbm, cnt_out, cidx, ones_v, zbuf, acc_sh, *ssems):
        cid = lax.axis_index("c")
        sid = lax.axis_index("s")
        wid = cid * NS + sid

        _fill_vmem(zbuf, ZR, jnp.zeros((16,), _f32))
        _fill_vmem(ones_v, C, jnp.ones((16,), _f32))
        _zero_chunks(zbuf, acc_sh, sid)
        pltpu.sync_copy(col3_hbm.at[wid], cidx)
        plsc.subcore_barrier()

        for b in range(NSS):
            pltpu.async_copy(ones_v, acc_sh.at[cidx.at[b]], ssems[b], add=True)

        def outer(o, carry):
            for b in range(NSS):
                t = o * NSS + b
                pltpu.make_async_copy(
                    ones_v, acc_sh.at[cidx.at[t]], ssems[b]).wait()

                @pl.when(t + NSS < NFULL)
                def _():
                    pltpu.async_copy(ones_v, acc_sh.at[cidx.at[t + NSS]],
                                     ssems[b], add=True)
            return carry

        lax.fori_loop(0, NFULL // NSS, outer, 0)
        for t in range(NFULL, CH):
            pltpu.sync_copy(ones_v, acc_sh.at[cidx.at[t]], add=True)
        plsc.subcore_barrier()
        _copy_chunks(acc_sh, cnt_out, cid, sid)

    return pl.kernel(
        body,
        out_type=jax.ShapeDtypeStruct((NC, N, D), _f32),
        mesh=mesh,
        scratch_types=[
            pltpu.VMEM((CH, C), jnp.int32),
            pltpu.VMEM((C, D), _f32),
            pltpu.VMEM((ZR, D), _f32),
            pltpu.VMEM_SHARED((N, D), _f32),
        ] + [pltpu.SemaphoreType.DMA] * NSS,
    )


def _inv_kernel(cnt):
    def body(c_ref, o_ref):
        c = c_ref[0, :, 0:1] + c_ref[1, :, 0:1]
        o_ref[...] = K_SH / jnp.maximum(c, 1.0)

    return pl.pallas_call(
        body, out_shape=jax.ShapeDtypeStruct((N, 1), _f32),
    )(cnt)


def _mm_in(x, w, b):
    def body(x_ref, w_ref, b_ref, o_ref):
        acc = lax.dot_general(x_ref[...], w_ref[...], (((1,), (1,)), ((), ())),
                              preferred_element_type=_f32)
        o_ref[...] = jnp.maximum(acc + b_ref[...], 0.0)

    return pl.pallas_call(
        body, out_shape=jax.ShapeDtypeStruct((N, H), _f32),
    )(x, w, b.reshape(1, H))


def _mm_layer(part, inv, w):
    def body(p_ref, i_ref, w_ref, o_ref):
        p = p_ref[0] + p_ref[1]
        h = lax.dot_general(p * i_ref[...], w_ref[...],
                            (((1,), (0,)), ((), ())),
                            preferred_element_type=_f32)
        o_ref[...] = jnp.maximum(h, 0.0)

    return pl.pallas_call(
        body, out_shape=jax.ShapeDtypeStruct((N, H), _f32),
    )(part, inv, w)


def _pool_mlp(part3, inv, wtp3, batch2d, w1, b1, w2, b2, w3, b3):
    def body(p_ref, i_ref, wtp_ref, bt_ref, w1_ref, b1_ref, w2_ref, b2_ref,
             w3_ref, b3_ref, out_ref, z_ref):
        p = p_ref[0] + p_ref[1]
        h3 = lax.dot_general(p * i_ref[...], wtp_ref[...],
                             (((1,), (0,)), ((), ())),
                             preferred_element_type=_f32)
        bt = bt_ref[...]
        groups = lax.broadcasted_iota(jnp.int32, (1, G), 1)
        oneh = (bt == groups).astype(_f32)
        zsum = lax.dot_general(oneh, h3, (((0,), (0,)), ((), ())),
                               preferred_element_type=_f32)
        ones = jnp.ones((N, 1), _f32)
        cnt = lax.dot_general(oneh, ones, (((0,), (0,)), ((), ())),
                              preferred_element_type=_f32)
        z = zsum / jnp.maximum(cnt, 1.0)
        h1 = jnp.maximum(
            lax.dot_general(z, w1_ref[...], (((1,), (1,)), ((), ())),
                            preferred_element_type=_f32) + b1_ref[...], 0.0)
        h2 = jnp.maximum(
            lax.dot_general(h1, w2_ref[...], (((1,), (1,)), ((), ())),
                            preferred_element_type=_f32) + b2_ref[...], 0.0)
        out = lax.dot_general(h2, w3_ref[...], (((1,), (1,)), ((), ())),
                              preferred_element_type=_f32) + b3_ref[...]
        out_ref[...] = out
        z_ref[...] = z

    return pl.pallas_call(
        body,
        out_shape=(jax.ShapeDtypeStruct((G, OUT), _f32),
                   jax.ShapeDtypeStruct((G, H), _f32)),
    )(part3, inv, wtp3, batch2d, w1, b1.reshape(1, H), w2, b2.reshape(1, H),
      w3, b3.reshape(1, OUT))


def kernel(x, edge_index, edge_attr, batch, pos, W_in, b_in,
           Wtp1, Wtp2, Wtp3, Wc1, bc1, Wc2, bc2, Wc3, bc3):
    row4 = edge_index[0].astype(jnp.int32).reshape(NW, CH // BI, BI, C)
    col4 = edge_index[1].astype(jnp.int32).reshape(NW, CH // BI, BI, C)
    col3 = edge_index[1].astype(jnp.int32).reshape(NW, CH, C)

    agg = _make_sc_agg()
    inv = _inv_kernel(_make_sc_cnt()(col3))
    h0 = _mm_in(x, W_in, b_in)

    part1 = agg(h0, row4, col4, inv)
    h1 = _mm_layer(part1, inv, Wtp1)
    part2 = agg(h1, row4, col4, inv)
    h2 = _mm_layer(part2, inv, Wtp2)
    part3 = agg(h2, row4, col4, inv)

    out, z = _pool_mlp(part3, inv, Wtp3,
                       batch.astype(jnp.int32).reshape(N, 1),
                       Wc1, bc1, Wc2, bc2, Wc3, bc3)
    return (out, z)


# speedup vs baseline: 10.9899x; 1.0075x over previous
import numpy as np
import jax
import jax.numpy as jnp
from jax import lax
from jax.experimental import pallas as pl
from jax.experimental.pallas import tpu as pltpu
from jax.experimental.pallas import tpu_sc as plsc

N = 10000
E = 320000
D = 128
H = 128
OUT = 10
G = 8

NC = 2
NS = 16
NW = NC * NS
EPW = E // NW
C = 80
CH = EPW // C
BI = 25
NB = 4
RCH = 400
NRCH = N // RCH
ZR = 40

K_SH = float(np.float32(0.28209479177387814) / np.float32(np.sqrt(np.float32(H))))

_f32 = jnp.float32


def _fill_vmem(ref, nrows, val16):
    def body(t, carry):
        ref[t // 8, pl.ds((t % 8) * 16, 16)] = val16
        return carry
    lax.fori_loop(0, nrows * 8, body, 0)


def _zero_chunks(zbuf, acc_sh, sid):
    def zero_chunk(base):
        for j in range(RCH // ZR):
            pltpu.sync_copy(zbuf, acc_sh.at[pl.ds(base + j * ZR, ZR)])

    zero_chunk(sid * RCH)

    @pl.when(sid < NRCH - NS)
    def _():
        zero_chunk((sid + NS) * RCH)


def _copy_chunks(acc_sh, out_hbm, cid, sid):
    def copy_chunk(base):
        sl = pl.ds(base, RCH)
        pltpu.sync_copy(acc_sh.at[sl], out_hbm.at[cid, sl])

    copy_chunk(sid * RCH)

    @pl.when(sid < NRCH - NS)
    def _():
        copy_chunk((sid + NS) * RCH)


def _make_sc_agg():
    mesh = plsc.VectorSubcoreMesh(core_axis_name="c", subcore_axis_name="s")

    NFULL = (BI // NB) * NB

    def body(h_hbm, row4_hbm, col4_hbm, dep_hbm, part_out,
             ridx, cidx, rows_v, acc_sh, *gsems):
        del dep_hbm
        cid = lax.axis_index("c")
        sid = lax.axis_index("s")
        wid = cid * NS + sid

        zslab = rows_v.at[0]
        _fill_vmem(zslab, C, jnp.zeros((16,), _f32))

        def zero_chunk(base):
            for j in range(RCH // C):
                pltpu.sync_copy(zslab, acc_sh.at[pl.ds(base + j * C, C)])

        zero_chunk(sid * RCH)

        @pl.when(sid < NRCH - NS)
        def _():
            zero_chunk((sid + NS) * RCH)
        plsc.subcore_barrier()

        def block(blk, carry):
            pltpu.sync_copy(row4_hbm.at[wid, blk], ridx)
            pltpu.sync_copy(col4_hbm.at[wid, blk], cidx)

            for b in range(NB):
                pltpu.async_copy(h_hbm.at[ridx.at[b]], rows_v.at[b], gsems[b])

            def outer(o, c2):
                for b in range(NB):
                    t = o * NB + b
                    pltpu.make_async_copy(
                        h_hbm.at[ridx.at[t]], rows_v.at[b], gsems[b]).wait()
                    pltpu.sync_copy(rows_v.at[b], acc_sh.at[cidx.at[t]],
                                    add=True)

                    @pl.when(t + NB < NFULL)
                    def _():
                        pltpu.async_copy(
                            h_hbm.at[ridx.at[t + NB]], rows_v.at[b], gsems[b])
                return c2

            lax.fori_loop(0, NFULL // NB, outer, 0)
            for t in range(NFULL, BI):
                pltpu.sync_copy(h_hbm.at[ridx.at[t]], rows_v.at[0])
                pltpu.sync_copy(rows_v.at[0], acc_sh.at[cidx.at[t]], add=True)
            return carry

        lax.fori_loop(0, CH // BI, block, 0)
        plsc.subcore_barrier()
        _copy_chunks(acc_sh, part_out, cid, sid)

    return pl.kernel(
        body,
        out_type=jax.ShapeDtypeStruct((NC, N, D), _f32),
        mesh=mesh,
        scratch_types=[
            pltpu.VMEM((BI, C), jnp.int32),
            pltpu.VMEM((BI, C), jnp.int32),
            pltpu.VMEM((NB, C, D), _f32),
            pltpu.VMEM_SHARED((N, D), _f32),
        ] + [pltpu.SemaphoreType.DMA] * NB,
    )


def _make_sc_cnt():
    mesh = plsc.VectorSubcoreMesh(core_axis_name="c", subcore_axis_name="s")

    NSS = 4
    NFULL = (CH // NSS) * NSS

    def body(col3_hbm, cnt_out, cidx, ones_v, zbuf, acc_sh, *ssems):
        cid = lax.axis_index("c")
        sid = lax.axis_index("s")
        wid = cid * NS + sid

        _fill_vmem(zbuf, ZR, jnp.zeros((16,), _f32))
        _fill_vmem(ones_v, C, jnp.ones((16,), _f32))
        _zero_chunks(zbuf, acc_sh, sid)
        pltpu.sync_copy(col3_hbm.at[wid], cidx)
        plsc.subcore_barrier()

        for b in range(NSS):
            pltpu.async_copy(ones_v, acc_sh.at[cidx.at[b]], ssems[b], add=True)

        def outer(o, carry):
            for b in range(NSS):
                t = o * NSS + b
                pltpu.make_async_copy(
                    ones_v, acc_sh.at[cidx.at[t]], ssems[b]).wait()

                @pl.when(t + NSS < NFULL)
                def _():
                    pltpu.async_copy(ones_v, acc_sh.at[cidx.at[t + NSS]],
                                     ssems[b], add=True)
            return carry

        lax.fori_loop(0, NFULL // NSS, outer, 0)
        for t in range(NFULL, CH):
            pltpu.sync_copy(ones_v, acc_sh.at[cidx.at[t]], add=True)
        plsc.subcore_barrier()
        _copy_chunks(acc_sh, cnt_out, cid, sid)

    return pl.kernel(
        body,
        out_type=jax.ShapeDtypeStruct((NC, N, D), _f32),
        mesh=mesh,
        scratch_types=[
            pltpu.VMEM((CH, C), jnp.int32),
            pltpu.VMEM((C, D), _f32),
            pltpu.VMEM((ZR, D), _f32),
            pltpu.VMEM_SHARED((N, D), _f32),
        ] + [pltpu.SemaphoreType.DMA] * NSS,
    )


def _inv_kernel(cnt):
    def body(c_ref, o_ref):
        c = c_ref[0, :, 0:1] + c_ref[1, :, 0:1]
        o_ref[...] = K_SH / jnp.maximum(c, 1.0)

    return pl.pallas_call(
        body, out_shape=jax.ShapeDtypeStruct((N, 1), _f32),
    )(cnt)


def _mm_in(x, w, b):
    def body(x_ref, w_ref, b_ref, o_ref):
        acc = lax.dot_general(x_ref[...], w_ref[...], (((1,), (1,)), ((), ())),
                              preferred_element_type=_f32)
        o_ref[...] = jnp.maximum(acc + b_ref[...], 0.0)

    return pl.pallas_call(
        body, out_shape=jax.ShapeDtypeStruct((N, H), _f32),
    )(x, w, b.reshape(1, H))


def _mm_layer(part, inv, w):
    def body(p_ref, i_ref, w_ref, o_ref):
        p = p_ref[0] + p_ref[1]
        h = lax.dot_general(p * i_ref[...], w_ref[...],
                            (((1,), (0,)), ((), ())),
                            preferred_element_type=_f32)
        o_ref[...] = jnp.maximum(h, 0.0)

    return pl.pallas_call(
        body, out_shape=jax.ShapeDtypeStruct((N, H), _f32),
    )(part, inv, w)


def _pool_mlp(part3, inv, wtp3, batch2d, w1, b1, w2, b2, w3, b3):
    def body(p_ref, i_ref, wtp_ref, bt_ref, w1_ref, b1_ref, w2_ref, b2_ref,
             w3_ref, b3_ref, out_ref, z_ref):
        p = p_ref[0] + p_ref[1]
        h3 = lax.dot_general(p * i_ref[...], wtp_ref[...],
                             (((1,), (0,)), ((), ())),
                             preferred_element_type=_f32)
        bt = bt_ref[...]
        groups = lax.broadcasted_iota(jnp.int32, (1, G), 1)
        oneh = (bt == groups).astype(_f32)
        zsum = lax.dot_general(oneh, h3, (((0,), (0,)), ((), ())),
                               preferred_element_type=_f32)
        ones = jnp.ones((N, 1), _f32)
        cnt = lax.dot_general(oneh, ones, (((0,), (0,)), ((), ())),
                              preferred_element_type=_f32)
        z = zsum / jnp.maximum(cnt, 1.0)
        h1 = jnp.maximum(
            lax.dot_general(z, w1_ref[...], (((1,), (1,)), ((), ())),
                            preferred_element_type=_f32) + b1_ref[...], 0.0)
        h2 = jnp.maximum(
            lax.dot_general(h1, w2_ref[...], (((1,), (1,)), ((), ())),
                            preferred_element_type=_f32) + b2_ref[...], 0.0)
        out = lax.dot_general(h2, w3_ref[...], (((1,), (1,)), ((), ())),
                              preferred_element_type=_f32) + b3_ref[...]
        out_ref[...] = out
        z_ref[...] = z

    return pl.pallas_call(
        body,
        out_shape=(jax.ShapeDtypeStruct((G, OUT), _f32),
                   jax.ShapeDtypeStruct((G, H), _f32)),
    )(part3, inv, wtp3, batch2d, w1, b1.reshape(1, H), w2, b2.reshape(1, H),
      w3, b3.reshape(1, OUT))


def kernel(x, edge_index, edge_attr, batch, pos, W_in, b_in,
           Wtp1, Wtp2, Wtp3, Wc1, bc1, Wc2, bc2, Wc3, bc3):
    row4 = edge_index[0].astype(jnp.int32).reshape(NW, CH // BI, BI, C)
    col4 = edge_index[1].astype(jnp.int32).reshape(NW, CH // BI, BI, C)
    col3 = edge_index[1].astype(jnp.int32).reshape(NW, CH, C)

    agg = _make_sc_agg()
    cnt = _make_sc_cnt()(col3)
    inv = _inv_kernel(cnt)
    h0 = _mm_in(x, W_in, b_in)

    part1 = agg(h0, row4, col4, cnt)
    h1 = _mm_layer(part1, inv, Wtp1)
    part2 = agg(h1, row4, col4, cnt)
    h2 = _mm_layer(part2, inv, Wtp2)
    part3 = agg(h2, row4, col4, cnt)

    out, z = _pool_mlp(part3, inv, Wtp3,
                       batch.astype(jnp.int32).reshape(N, 1),
                       Wc1, bc1, Wc2, bc2, Wc3, bc3)
    return (out, z)
